# baseline (speedup 1.0000x reference)
import functools

import jax
import jax.numpy as jnp
from jax import lax
from jax.experimental import pallas as pl
from jax.experimental.pallas import tpu as pltpu
from jax.experimental.pallas import tpu_sc as plsc

CH = 64
Q = 6


def _sc_scatter(prev_pad, srcs, dsts, n_nodes, d, nc, ns, g0, g1):
  grp = Q * CH
  blk = 80
  nblocks = n_nodes // blk
  assert n_nodes % blk == 0 and blk % 16 == 0

  mesh = plsc.VectorSubcoreMesh(core_axis_name="c", subcore_axis_name="s")

  scratch = (
      [pltpu.VMEM((CH,), jnp.int32) for _ in range(Q)]
      + [pltpu.VMEM((CH,), jnp.int32) for _ in range(Q)]
      + [pltpu.VMEM((CH, d), jnp.float32) for _ in range(Q)]
      + [pltpu.VMEM_SHARED((n_nodes, d), jnp.float32)]
      + [pltpu.SemaphoreType.DMA for _ in range(4 * Q)]
  )

  @functools.partial(
      pl.kernel,
      out_type=jax.ShapeDtypeStruct((nc * n_nodes, d), jnp.float32),
      mesh=mesh,
      scratch_types=scratch,
  )
  def body(prev_hbm, srcs_hbm, dsts_hbm, out_hbm, *scr):
    sidx = scr[0:Q]
    didx = scr[Q:2 * Q]
    rows = scr[2 * Q:3 * Q]
    acc = scr[3 * Q]
    isems = scr[3 * Q + 1:3 * Q + 1 + Q]
    jsems = scr[3 * Q + 1 + Q:3 * Q + 1 + 2 * Q]
    gsems = scr[3 * Q + 1 + 2 * Q:3 * Q + 1 + 3 * Q]
    ssems = scr[3 * Q + 1 + 3 * Q:3 * Q + 1 + 4 * Q]

    cid = lax.axis_index("c")
    sid = lax.axis_index("s")
    tile_start = jnp.where(cid == 0, sid * g0, ns * g0 + sid * g1) * grp
    my_groups = jnp.where(cid == 0, g0, g1)
    my_nblk = (nblocks - 1 - sid) // ns + 1

    zvec = jnp.zeros((16,), jnp.float32)
    for r in range(8):
      for c in range(d // 16):
        rows[0][r, pl.ds(c * 16, 16)] = zvec

    def zero_body(j, carry):
      base = (sid + j * ns) * blk
      for k in range(blk // 8):
        pltpu.sync_copy(rows[0].at[pl.ds(0, 8)],
                        acc.at[pl.ds(base + k * 8, 8)])
      return carry

    lax.fori_loop(0, my_nblk, zero_body, 0)
    plsc.subcore_barrier()

    def drain_scatters():
      for k in range(Q):
        pltpu.make_async_copy(rows[k], acc.at[didx[k]], ssems[k]).wait()

    def group_body(g, carry):
      @pl.when(g > 0)
      def _():
        drain_scatters()

      base = tile_start + g * grp
      idesc = [None] * Q
      jdesc = [None] * Q
      gdesc = [None] * Q
      for k in range(Q):
        off = base + k * CH
        idesc[k] = pltpu.async_copy(
            srcs_hbm.at[pl.ds(off, CH)], sidx[k], isems[k])
        jdesc[k] = pltpu.async_copy(
            dsts_hbm.at[pl.ds(off, CH)], didx[k], jsems[k])
      for k in range(Q):
        idesc[k].wait()
        gdesc[k] = pltpu.async_copy(prev_hbm.at[sidx[k]], rows[k], gsems[k])
      for k in range(Q):
        gdesc[k].wait()
        jdesc[k].wait()
        pltpu.async_copy(rows[k], acc.at[didx[k]], ssems[k], add=True)
      return carry

    lax.fori_loop(0, my_groups, group_body, 0)
    drain_scatters()
    plsc.subcore_barrier()

    def wr_body(j, carry):
      base = (sid + j * ns) * blk
      pltpu.sync_copy(acc.at[pl.ds(base, blk)],
                      out_hbm.at[pl.ds(cid * n_nodes + base, blk)])
      return carry

    lax.fori_loop(0, my_nblk, wr_body, 0)

  return body(prev_pad, srcs, dsts)


def _tc_finish(partials, nodef, edgef, w2, n_nodes, d):
  bn = 1000
  grid = n_nodes // bn

  def body(p0_ref, p1_ref, nf_ref, ef_ref, w2_ref, out_ref):
    nbr = p0_ref[...] + p1_ref[...]
    x2 = lax.dot_general(
        nbr, w2_ref[...],
        dimension_numbers=(((1,), (1,)), ((), ())),
        preferred_element_type=jnp.float32,
    )
    x = nf_ref[...] + ef_ref[...] + x2
    out_ref[...] = jnp.where(x >= 0, x, 0.01 * x)

  row_spec = pl.BlockSpec((bn, d), lambda i: (i, 0))
  p1_spec = pl.BlockSpec((bn, d), lambda i: (grid + i, 0))
  return pl.pallas_call(
      body,
      grid=(grid,),
      in_specs=[row_spec, p1_spec, row_spec, row_spec,
                pl.BlockSpec((d, d), lambda i: (0, 0))],
      out_specs=row_spec,
      out_shape=jax.ShapeDtypeStruct((n_nodes, d), jnp.float32),
  )(partials, partials, nodef, edgef, w2)


def kernel(prev_embeddings, edges_ij, node_features_embeddings, edge_features_embeddings, W2):
  b, n, d = prev_embeddings.shape
  e = edges_ij.shape[0]

  info = plsc.get_sparse_core_info()
  nc, ns = info.num_cores, info.num_subcores
  nw = nc * ns

  grp = Q * CH
  g_tot = -(-2 * e // (ns * grp))
  g0 = (61 * g_tot) // 100
  g1 = g_tot - g0
  pe = ns * g_tot * grp

  u = edges_ij[:, 0]
  v = edges_ij[:, 1]
  pad = pe - 2 * e
  srcs = jnp.concatenate([v, u, jnp.full((pad,), n, jnp.int32)])
  dsts = jnp.concatenate([u, v, jnp.zeros((pad,), jnp.int32)])
  prev_pad = jnp.concatenate(
      [prev_embeddings[0], jnp.zeros((8, d), jnp.float32)], axis=0)

  partials = _sc_scatter(prev_pad, srcs, dsts, n, d, nc, ns, g0, g1)

  out = _tc_finish(partials, node_features_embeddings[0],
                   edge_features_embeddings[0], W2, n, d)
  return out.reshape(b, n, d)


# speedup vs baseline: 1.2245x; 1.2245x over previous
import functools

import jax
import jax.numpy as jnp
from jax import lax
from jax.experimental import pallas as pl
from jax.experimental.pallas import tpu as pltpu
from jax.experimental.pallas import tpu_sc as plsc

CH = 64
Q = 5


def _sc_scatter(prev_pad, srcs, dsts, n_nodes, d, nc, ns, g0, g1):
  grp = Q * CH
  blk = 80
  nblocks = n_nodes // blk
  assert n_nodes % blk == 0 and blk % 16 == 0

  mesh = plsc.VectorSubcoreMesh(core_axis_name="c", subcore_axis_name="s")

  scratch = (
      [pltpu.VMEM((CH,), jnp.int32) for _ in range(Q)]
      + [pltpu.VMEM((CH,), jnp.int32) for _ in range(Q)]
      + [pltpu.VMEM((CH, d), jnp.float32) for _ in range(Q)]
      + [pltpu.VMEM((16, d), jnp.float32)]
      + [pltpu.VMEM_SHARED((n_nodes, d), jnp.float32)]
      + [pltpu.SemaphoreType.DMA for _ in range(4 * Q)]
  )

  @functools.partial(
      pl.kernel,
      out_type=jax.ShapeDtypeStruct((nc * n_nodes, d), jnp.float32),
      mesh=mesh,
      scratch_types=scratch,
  )
  def body(prev_hbm, srcs_hbm, dsts_hbm, out_hbm, *scr):
    sidx = scr[0:Q]
    didx = scr[Q:2 * Q]
    rows = scr[2 * Q:3 * Q]
    zbuf = scr[3 * Q]
    acc = scr[3 * Q + 1]
    isems = scr[3 * Q + 2:3 * Q + 2 + Q]
    jsems = scr[3 * Q + 2 + Q:3 * Q + 2 + 2 * Q]
    gsems = scr[3 * Q + 2 + 2 * Q:3 * Q + 2 + 3 * Q]
    ssems = scr[3 * Q + 2 + 3 * Q:3 * Q + 2 + 4 * Q]

    cid = lax.axis_index("c")
    sid = lax.axis_index("s")
    tile_start = jnp.where(cid == 0, sid * g0, ns * g0 + sid * g1) * grp
    my_groups = jnp.where(cid == 0, g0, g1)
    my_nblk = (nblocks - 1 - sid) // ns + 1

    zvec = jnp.zeros((16,), jnp.float32)
    for r in range(16):
      for c in range(d // 16):
        zbuf[r, pl.ds(c * 16, 16)] = zvec

    def zero_body(j, carry):
      base = (sid + j * ns) * blk
      for k in range(blk // 16):
        pltpu.sync_copy(zbuf, acc.at[pl.ds(base + k * 16, 16)])
      return carry

    lax.fori_loop(0, my_nblk, zero_body, 0)
    plsc.subcore_barrier()

    def drain_scatters():
      for k in range(Q):
        pltpu.make_async_copy(rows[k], acc.at[didx[k]], ssems[k]).wait()

    def group_body(g, carry):
      @pl.when(g > 0)
      def _():
        drain_scatters()

      base = tile_start + g * grp
      idesc = [None] * Q
      jdesc = [None] * Q
      gdesc = [None] * Q
      for k in range(Q):
        off = base + k * CH
        idesc[k] = pltpu.async_copy(
            srcs_hbm.at[pl.ds(off, CH)], sidx[k], isems[k])
        jdesc[k] = pltpu.async_copy(
            dsts_hbm.at[pl.ds(off, CH)], didx[k], jsems[k])
      for k in range(Q):
        idesc[k].wait()
        gdesc[k] = pltpu.async_copy(prev_hbm.at[sidx[k]], rows[k], gsems[k])
      for k in range(Q):
        gdesc[k].wait()
        jdesc[k].wait()
        pltpu.async_copy(rows[k], acc.at[didx[k]], ssems[k], add=True)
      return carry

    lax.fori_loop(0, my_groups, group_body, 0)
    drain_scatters()
    plsc.subcore_barrier()

    def wr_body(j, carry):
      base = (sid + j * ns) * blk
      pltpu.sync_copy(acc.at[pl.ds(base, blk)],
                      out_hbm.at[pl.ds(cid * n_nodes + base, blk)])
      return carry

    lax.fori_loop(0, my_nblk, wr_body, 0)

  return body(prev_pad, srcs, dsts)


def _tc_finish(partials, nodef, edgef, w2, n_nodes, d):
  bn = 1000
  grid = n_nodes // bn

  def body(p0_ref, p1_ref, nf_ref, ef_ref, w2_ref, out_ref):
    nbr = p0_ref[...] + p1_ref[...]
    x2 = lax.dot_general(
        nbr, w2_ref[...],
        dimension_numbers=(((1,), (1,)), ((), ())),
        preferred_element_type=jnp.float32,
    )
    x = nf_ref[...] + ef_ref[...] + x2
    out_ref[...] = jnp.where(x >= 0, x, 0.01 * x)

  row_spec = pl.BlockSpec((bn, d), lambda i: (i, 0))
  p1_spec = pl.BlockSpec((bn, d), lambda i: (grid + i, 0))
  return pl.pallas_call(
      body,
      grid=(grid,),
      in_specs=[row_spec, p1_spec, row_spec, row_spec,
                pl.BlockSpec((d, d), lambda i: (0, 0))],
      out_specs=row_spec,
      out_shape=jax.ShapeDtypeStruct((n_nodes, d), jnp.float32),
  )(partials, partials, nodef, edgef, w2)


def kernel(prev_embeddings, edges_ij, node_features_embeddings, edge_features_embeddings, W2):
  b, n, d = prev_embeddings.shape
  e = edges_ij.shape[0]

  info = plsc.get_sparse_core_info()
  nc, ns = info.num_cores, info.num_subcores
  nw = nc * ns

  grp = Q * CH
  g_tot = -(-2 * e // (ns * grp))
  g0 = (61 * g_tot) // 100
  g1 = g_tot - g0
  pe = ns * g_tot * grp

  u = edges_ij[:, 0]
  v = edges_ij[:, 1]
  pad = pe - 2 * e
  srcs = jnp.concatenate([v, u, jnp.full((pad,), n, jnp.int32)])
  dsts = jnp.concatenate([u, v, jnp.zeros((pad,), jnp.int32)])
  prev_pad = jnp.concatenate(
      [prev_embeddings[0], jnp.zeros((8, d), jnp.float32)], axis=0)

  partials = _sc_scatter(prev_pad, srcs, dsts, n, d, nc, ns, g0, g1)

  out = _tc_finish(partials, node_features_embeddings[0],
                   edge_features_embeddings[0], W2, n, d)
  return out.reshape(b, n, d)
